# Initial kernel scaffold; baseline (speedup 1.0000x reference)
import jax
import jax.numpy as jnp
from jax.experimental import pallas as pl


def kernel(features, feat_out, nodes, geo_neighbors, weight):
    raise NotImplementedError("write your pallas kernel here")


# speedup vs baseline: 1.1562x; 1.1562x over previous
import functools

import jax
import jax.numpy as jnp
from jax import lax
from jax.experimental import pallas as pl
from jax.experimental.pallas import tpu as pltpu
from jax.experimental.pallas import tpu_sc as plsc

_NW = 32
_LANES = 16


def _sc_gather_mean(nodes, geo, k, feat2d):
    b_total = nodes.shape[0]
    n_rows, k_pad = geo.shape
    td = feat2d.shape[1]
    bpw = b_total // _NW
    chunks = td // _LANES
    scale = 1.0 / k

    mesh = plsc.VectorSubcoreMesh(core_axis_name="c", subcore_axis_name="s")

    @functools.partial(
        pl.kernel,
        out_type=jax.ShapeDtypeStruct((b_total, td), jnp.float32),
        mesh=mesh,
        scratch_types=[
            pltpu.VMEM((bpw,), jnp.int32),
            pltpu.VMEM((bpw, k_pad), jnp.int32),
            pltpu.VMEM((k, td), jnp.float32),
            pltpu.VMEM((k, td), jnp.float32),
            pltpu.VMEM((1, td), jnp.float32),
            pltpu.VMEM((1, td), jnp.float32),
            pltpu.SemaphoreType.DMA,
            pltpu.SemaphoreType.DMA,
            pltpu.SemaphoreType.DMA,
            pltpu.SemaphoreType.DMA,
        ],
    )
    def sc_kernel(nodes_hbm, geo_hbm, feat_hbm, out_hbm,
                  nodes_v, geo_v, rows0, rows1, out0, out1,
                  sem_g0, sem_g1, sem_o0, sem_o1):
        w = lax.axis_index("s") * 2 + lax.axis_index("c")
        base = w * bpw

        pltpu.sync_copy(nodes_hbm.at[pl.ds(base, bpw)], nodes_v)
        pltpu.async_copy(geo_hbm.at[nodes_v], geo_v, sem_g0).wait()

        rows = (rows0, rows1)
        outs = (out0, out1)
        sems_g = (sem_g0, sem_g1)
        sems_o = (sem_o0, sem_o1)

        def fire_gather(i, b):
            pltpu.async_copy(feat_hbm.at[geo_v.at[i, pl.ds(0, k)]],
                             rows[b], sems_g[b])

        fire_gather(0, 0)
        fire_gather(1, 1)

        def process(i, b):
            pltpu.make_async_copy(
                feat_hbm.at[geo_v.at[i, pl.ds(0, k)]], rows[b],
                sems_g[b]).wait()

            @pl.when(i >= 2)
            def _():
                pltpu.make_async_copy(
                    outs[b], out_hbm.at[pl.ds(base + i - 2, 1)],
                    sems_o[b]).wait()

            def chunk(j, carry):
                o = j * _LANES
                acc = rows[b][0, pl.ds(o, _LANES)]
                for r in range(1, k):
                    acc = acc + rows[b][r, pl.ds(o, _LANES)]
                outs[b][0, pl.ds(o, _LANES)] = acc * scale
                return carry

            lax.fori_loop(0, chunks, chunk, 0)

            pltpu.async_copy(outs[b], out_hbm.at[pl.ds(base + i, 1)],
                             sems_o[b])

            @pl.when(i + 2 < bpw)
            def _():
                fire_gather(i + 2, b)

        def outer(g, carry):
            process(2 * g, 0)
            process(2 * g + 1, 1)
            return carry

        lax.fori_loop(0, bpw // 2, outer, 0)

        pltpu.make_async_copy(
            out0, out_hbm.at[pl.ds(base + bpw - 2, 1)], sem_o0).wait()
        pltpu.make_async_copy(
            out1, out_hbm.at[pl.ds(base + bpw - 1, 1)], sem_o1).wait()

    return sc_kernel(nodes, geo, feat2d)


def _tc_matmul_relu(x1, x2, weight, block_m=512):
    m, d = x1.shape
    e = weight.shape[0]

    def body(x1_ref, x2_ref, w_ref, o_ref):
        w1 = w_ref[:, :d]
        w2 = w_ref[:, d:]
        dn = (((1,), (1,)), ((), ()))
        acc = lax.dot_general(x1_ref[...], w1, dn,
                              preferred_element_type=jnp.float32)
        acc = acc + lax.dot_general(x2_ref[...], w2, dn,
                                    preferred_element_type=jnp.float32)
        o_ref[...] = jnp.maximum(acc, 0.0)

    return pl.pallas_call(
        body,
        grid=(m // block_m,),
        in_specs=[
            pl.BlockSpec((block_m, d), lambda i: (i, 0)),
            pl.BlockSpec((block_m, d), lambda i: (i, 0)),
            pl.BlockSpec((e, 2 * d), lambda i: (0, 0)),
        ],
        out_specs=pl.BlockSpec((block_m, e), lambda i: (i, 0)),
        out_shape=jax.ShapeDtypeStruct((m, e), jnp.float32),
    )(x1, x2, weight)


def kernel(features, feat_out, nodes, geo_neighbors, weight):
    b, t, d = features.shape
    n = feat_out.shape[0]
    e = weight.shape[0]

    nodes_i = nodes.astype(jnp.int32)
    k = geo_neighbors.shape[1]
    geo_pad = jnp.pad(geo_neighbors, ((0, 0), (0, 128 - k)))
    feat2d = feat_out.reshape(n, t * d)
    neigh = _sc_gather_mean(nodes_i, geo_pad, k, feat2d)

    x1 = features.reshape(b * t, d)
    x2 = neigh.reshape(b * t, d)
    out = _tc_matmul_relu(x1, x2, weight)
    return out.reshape(b, t, e)
